# Initial kernel scaffold; baseline (speedup 1.0000x reference)
import jax
import jax.numpy as jnp
from jax.experimental import pallas as pl


def kernel(features, coords):
    raise NotImplementedError("write your pallas kernel here")


# speedup vs baseline: 4.6802x; 4.6802x over previous
import functools

import jax
import jax.numpy as jnp
import numpy as np
from jax import lax
from jax.experimental import pallas as pl
from jax.experimental.pallas import tpu as pltpu
from jax.experimental.pallas import tpu_sc as plsc

RES = 32
NVOX = RES ** 3

_B, _C, _N = 8, 16, 100000
_CH = 10000
_NCHUNK = _N // _CH
_GRP = _CH // 16
_UNITS = _B * _C + _B


def _f32(x):
    return np.float32(x)


def _atan_poly(v):
    z = v * v
    p = _f32(8.05374449538e-2) * z - _f32(1.38776856032e-1)
    p = p * z + _f32(1.99777106478e-1)
    p = p * z - _f32(3.33329491539e-1)
    return p * z * v + v


def _atan01(a):
    big = a > _f32(0.4142135623730951)
    t = jnp.where(big, (a - _f32(1.0)) / (a + _f32(1.0)), a)
    r = _atan_poly(t)
    return jnp.where(big, _f32(np.pi / 4) + r, r)


def _atan2(y, x):
    ax = jnp.abs(x)
    ay = jnp.abs(y)
    mx = jnp.maximum(ax, ay)
    mn = jnp.minimum(ax, ay)
    a = mn / jnp.where(mx > _f32(0.0), mx, _f32(1.0))
    r = _atan01(a)
    r = jnp.where(ay > ax, _f32(np.pi / 2) - r, r)
    r = jnp.where(x < _f32(0.0), _f32(np.pi) - r, r)
    return jnp.where(y < _f32(0.0), -r, r)


def _acos(w):
    s = jnp.sqrt(jnp.maximum((_f32(1.0) - w) * (_f32(1.0) + w), _f32(0.0)))
    return _atan2(s, w)


def _prep_body(c_ref, nc_ref, inds_ref):
    c = c_ref[0]
    n = c.shape[1]
    mean = jnp.sum(c, axis=1, keepdims=True) * _f32(1.0 / n)
    d = c - mean
    x = d[0:1, :]
    y = d[1:2, :]
    z = d[2:3, :]
    nsq = x * x + y * y + z * z
    max_norm = jnp.sqrt(jnp.max(nsq))
    inv = _f32(1.0) / (max_norm + _f32(1e-20))
    nc = d * inv
    nc_ref[0] = nc
    xn = nc[0:1, :]
    yn = nc[1:2, :]
    zn = nc[2:3, :]
    rho = jnp.sqrt(xn * xn + yn * yn + zn * zn)
    w = jnp.clip(zn / (rho + _f32(1e-20)), _f32(-1.0), _f32(1.0))
    theta = _acos(w)
    phi = _atan2(yn, xn)
    rho_bin = jnp.clip((rho * _f32(RES)).astype(jnp.int32), 0, RES - 1)
    theta_bin = jnp.clip(
        (theta / _f32(np.pi) * _f32(RES)).astype(jnp.int32), 0, RES - 1)
    phi_bin = jnp.clip(
        ((phi + _f32(np.pi)) / _f32(2.0 * np.pi) * _f32(RES)).astype(jnp.int32),
        0, RES - 1)
    inds_ref[0] = rho_bin * (RES * RES) + theta_bin * RES + phi_bin


def _prep(coords):
    b, _, n = coords.shape
    return pl.pallas_call(
        _prep_body,
        grid=(b,),
        in_specs=[pl.BlockSpec((1, 3, n), lambda i: (i, 0, 0))],
        out_specs=[
            pl.BlockSpec((1, 3, n), lambda i: (i, 0, 0)),
            pl.BlockSpec((1, 1, n), lambda i: (i, 0, 0)),
        ],
        out_shape=[
            jax.ShapeDtypeStruct((b, 3, n), jnp.float32),
            jax.ShapeDtypeStruct((b, 1, n), jnp.int32),
        ],
    )(coords)


@functools.cache
def _get_sc_scatter():
    return functools.partial(
        pl.kernel,
        mesh=plsc.VectorSubcoreMesh(core_axis_name="c", subcore_axis_name="s"),
        out_type=[
            jax.ShapeDtypeStruct((_B * _C * NVOX,), jnp.float32),
            jax.ShapeDtypeStruct((_B * NVOX,), jnp.float32),
        ],
        scratch_types=[
            pltpu.VMEM((NVOX,), jnp.float32),
            pltpu.VMEM((_CH,), jnp.int32),
            pltpu.VMEM((_CH,), jnp.float32),
        ],
        compiler_params=pltpu.CompilerParams(needs_layout_passes=False),
    )(_sc_scatter_body)


def _sc_scatter_body(feat_hbm, idx_hbm, sums_hbm, cnt_hbm, acc, ibuf, fbuf):
    wid = lax.axis_index("s") * 2 + lax.axis_index("c")
    ones = jnp.full((16,), 1.0, jnp.float32)
    zeros = jnp.zeros((16,), jnp.float32)

    def zero_acc():
        def zbody(i, carry):
            acc[pl.ds(i * 16, 16)] = zeros
            return carry
        lax.fori_loop(0, NVOX // 16, zbody, 0)

    for k in range((_UNITS + 31) // 32):
        u = wid + 32 * k

        @pl.when(u < _B * _C)
        def _():
            zero_acc()
            b = u // _C

            def cbody(ci, carry):
                st = ci * _CH
                pltpu.sync_copy(idx_hbm.at[pl.ds(b * _N + st, _CH)], ibuf)
                pltpu.sync_copy(feat_hbm.at[pl.ds(u * _N + st, _CH)], fbuf)

                def gbody(j, carry2):
                    iv = ibuf[pl.ds(j * 16, 16)]
                    fv = fbuf[pl.ds(j * 16, 16)]
                    plsc.addupdate_scatter(acc, [iv], fv)
                    return carry2
                lax.fori_loop(0, _GRP, gbody, 0)
                return carry
            lax.fori_loop(0, _NCHUNK, cbody, 0)
            pltpu.sync_copy(acc, sums_hbm.at[pl.ds(u * NVOX, NVOX)])

        @pl.when((u >= _B * _C) & (u < _UNITS))
        def _():
            zero_acc()
            b = u - _B * _C

            def cbody(ci, carry):
                st = ci * _CH
                pltpu.sync_copy(idx_hbm.at[pl.ds(b * _N + st, _CH)], ibuf)

                def gbody(j, carry2):
                    iv = ibuf[pl.ds(j * 16, 16)]
                    plsc.addupdate_scatter(acc, [iv], ones)
                    return carry2
                lax.fori_loop(0, _GRP, gbody, 0)
                return carry
            lax.fori_loop(0, _NCHUNK, cbody, 0)
            pltpu.sync_copy(acc, cnt_hbm.at[pl.ds(b * NVOX, NVOX)])


def _fin_body(s_ref, c_ref, o_ref):
    s = s_ref[0]
    cnt = c_ref[0]
    o_ref[0] = s / jnp.maximum(cnt, _f32(1.0))


def _finalize(sums, cnt):
    b, c, v = sums.shape
    return pl.pallas_call(
        _fin_body,
        grid=(b,),
        in_specs=[
            pl.BlockSpec((1, c, v), lambda i: (i, 0, 0)),
            pl.BlockSpec((1, 1, v), lambda i: (i, 0, 0)),
        ],
        out_specs=pl.BlockSpec((1, c, v), lambda i: (i, 0, 0)),
        out_shape=jax.ShapeDtypeStruct((b, c, v), jnp.float32),
    )(sums, cnt)


def kernel(features, coords):
    b, c, n = features.shape
    assert (b, c, n) == (_B, _C, _N), "kernel compiled for fixed shapes"
    coords = lax.stop_gradient(coords)
    norm_coords, inds3 = _prep(coords)
    sums, cnt = _get_sc_scatter()(
        features.reshape(b * c * n), inds3.reshape(b * n))
    out = _finalize(sums.reshape(b, c, NVOX), cnt.reshape(b, 1, NVOX))
    inds = lax.stop_gradient(inds3.reshape(b, n))
    return (out.reshape(b, c, RES, RES, RES), inds, norm_coords)


# speedup vs baseline: 5.9122x; 1.2633x over previous
import functools

import jax
import jax.numpy as jnp
import numpy as np
from jax import lax
from jax.experimental import pallas as pl
from jax.experimental.pallas import tpu as pltpu
from jax.experimental.pallas import tpu_sc as plsc

RES = 32
NVOX = RES ** 3

_B, _C, _N = 8, 16, 100000
_CH = 10000
_NCHUNK = _N // _CH
_GRP = _CH // 16
_UNITS = _B * _C + _B


def _f32(x):
    return np.float32(x)


def _atan_poly(v):
    z = v * v
    p = _f32(8.05374449538e-2) * z - _f32(1.38776856032e-1)
    p = p * z + _f32(1.99777106478e-1)
    p = p * z - _f32(3.33329491539e-1)
    return p * z * v + v


def _atan01(a):
    big = a > _f32(0.4142135623730951)
    t = jnp.where(big, (a - _f32(1.0)) / (a + _f32(1.0)), a)
    r = _atan_poly(t)
    return jnp.where(big, _f32(np.pi / 4) + r, r)


def _atan2(y, x):
    ax = jnp.abs(x)
    ay = jnp.abs(y)
    mx = jnp.maximum(ax, ay)
    mn = jnp.minimum(ax, ay)
    a = mn / jnp.where(mx > _f32(0.0), mx, _f32(1.0))
    r = _atan01(a)
    r = jnp.where(ay > ax, _f32(np.pi / 2) - r, r)
    r = jnp.where(x < _f32(0.0), _f32(np.pi) - r, r)
    return jnp.where(y < _f32(0.0), -r, r)


def _acos(w):
    s = jnp.sqrt(jnp.maximum((_f32(1.0) - w) * (_f32(1.0) + w), _f32(0.0)))
    return _atan2(s, w)


def _prep_body(c_ref, nc_ref, inds_ref):
    c = c_ref[0]
    n = c.shape[1]
    mean = jnp.sum(c, axis=1, keepdims=True) * _f32(1.0 / n)
    d = c - mean
    x = d[0:1, :]
    y = d[1:2, :]
    z = d[2:3, :]
    nsq = x * x + y * y + z * z
    max_norm = jnp.sqrt(jnp.max(nsq))
    inv = _f32(1.0) / (max_norm + _f32(1e-20))
    nc = d * inv
    nc_ref[0] = nc
    xn = nc[0:1, :]
    yn = nc[1:2, :]
    zn = nc[2:3, :]
    rho = jnp.sqrt(xn * xn + yn * yn + zn * zn)
    w = jnp.clip(zn / (rho + _f32(1e-20)), _f32(-1.0), _f32(1.0))
    theta = _acos(w)
    phi = _atan2(yn, xn)
    rho_bin = jnp.clip((rho * _f32(RES)).astype(jnp.int32), 0, RES - 1)
    theta_bin = jnp.clip(
        (theta / _f32(np.pi) * _f32(RES)).astype(jnp.int32), 0, RES - 1)
    phi_bin = jnp.clip(
        ((phi + _f32(np.pi)) / _f32(2.0 * np.pi) * _f32(RES)).astype(jnp.int32),
        0, RES - 1)
    inds_ref[0] = rho_bin * (RES * RES) + theta_bin * RES + phi_bin


def _prep(coords):
    b, _, n = coords.shape
    return pl.pallas_call(
        _prep_body,
        grid=(b,),
        in_specs=[pl.BlockSpec((1, 3, n), lambda i: (i, 0, 0))],
        out_specs=[
            pl.BlockSpec((1, 3, n), lambda i: (i, 0, 0)),
            pl.BlockSpec((1, 1, n), lambda i: (i, 0, 0)),
        ],
        out_shape=[
            jax.ShapeDtypeStruct((b, 3, n), jnp.float32),
            jax.ShapeDtypeStruct((b, 1, n), jnp.int32),
        ],
    )(coords)


@functools.cache
def _get_sc_scatter():
    return functools.partial(
        pl.kernel,
        mesh=plsc.VectorSubcoreMesh(core_axis_name="c", subcore_axis_name="s"),
        out_type=[
            jax.ShapeDtypeStruct((_B * _C * NVOX,), jnp.float32),
            jax.ShapeDtypeStruct((_B * NVOX,), jnp.float32),
        ],
        scratch_types=[
            pltpu.VMEM((NVOX,), jnp.float32),
            pltpu.VMEM((_CH,), jnp.int32),
            pltpu.VMEM((_CH,), jnp.int32),
            pltpu.VMEM((_CH,), jnp.float32),
            pltpu.VMEM((_CH,), jnp.float32),
            pltpu.SemaphoreType.DMA,
            pltpu.SemaphoreType.DMA,
            pltpu.SemaphoreType.DMA,
        ],
        compiler_params=pltpu.CompilerParams(needs_layout_passes=False),
    )(_sc_scatter_body)


_UNROLL = 5
_ZUNROLL = 16


def _sc_scatter_body(feat_hbm, idx_hbm, sums_hbm, cnt_hbm,
                     acc, ibuf0, ibuf1, fbuf0, fbuf1, s0, s1, sw):
    wid = lax.axis_index("s") * 2 + lax.axis_index("c")
    ones = jnp.full((16,), 1.0, jnp.float32)
    zeros = jnp.zeros((16,), jnp.float32)
    ibufs, fbufs, sems = (ibuf0, ibuf1), (fbuf0, fbuf1), (s0, s1)

    def zero_acc():
        def zbody(i, carry):
            for t in range(_ZUNROLL):
                acc[pl.ds(i * (16 * _ZUNROLL) + t * 16, 16)] = zeros
            return carry
        lax.fori_loop(0, NVOX // (16 * _ZUNROLL), zbody, 0)

    def start_chunk(idx_base, feat_base, ci, p, with_feat):
        st = ci * _CH
        pltpu.async_copy(
            idx_hbm.at[pl.ds(idx_base + st, _CH)], ibufs[p], sems[p])
        if with_feat:
            pltpu.async_copy(
                feat_hbm.at[pl.ds(feat_base + st, _CH)], fbufs[p], sems[p])

    def wait_chunk(p, with_feat):
        pltpu.make_async_copy(
            idx_hbm.at[pl.ds(0, _CH)], ibufs[p], sems[p]).wait()
        if with_feat:
            pltpu.make_async_copy(
                feat_hbm.at[pl.ds(0, _CH)], fbufs[p], sems[p]).wait()

    def scatter_chunk(p, with_feat):
        ib, fb = ibufs[p], fbufs[p]

        def gbody(j, carry):
            for t in range(_UNROLL):
                off = j * (16 * _UNROLL) + t * 16
                iv = ib[pl.ds(off, 16)]
                fv = fb[pl.ds(off, 16)] if with_feat else ones
                plsc.addupdate_scatter(acc, [iv], fv)
            return carry
        lax.fori_loop(0, _GRP // _UNROLL, gbody, 0)

    def wait_write():
        pltpu.make_async_copy(
            sums_hbm.at[pl.ds(0, NVOX)], acc, sw).wait()

    def run_unit(k, idx_base, feat_base, out_ref, out_off, with_feat):
        start_chunk(idx_base, feat_base, 0, 0, with_feat)
        if k > 0:
            wait_write()
        zero_acc()

        def pair_body(i, carry):
            start_chunk(idx_base, feat_base, 2 * i + 1, 1, with_feat)
            wait_chunk(0, with_feat)
            scatter_chunk(0, with_feat)

            @pl.when(2 * i + 2 < _NCHUNK)
            def _():
                start_chunk(idx_base, feat_base, 2 * i + 2, 0, with_feat)
            wait_chunk(1, with_feat)
            scatter_chunk(1, with_feat)
            return carry
        lax.fori_loop(0, _NCHUNK // 2, pair_body, 0)
        pltpu.async_copy(acc, out_ref.at[pl.ds(out_off, NVOX)], sw)

    for k in range((_UNITS + 31) // 32):
        u = wid + 32 * k

        @pl.when(u < _B * _C)
        def _():
            b = u // _C
            run_unit(k, b * _N, u * _N, sums_hbm, u * NVOX, True)

        @pl.when((u >= _B * _C) & (u < _UNITS))
        def _():
            b = u - _B * _C
            run_unit(k, b * _N, 0, cnt_hbm, b * NVOX, False)

    wait_write()


def _fin_body(s_ref, c_ref, o_ref):
    s = s_ref[0]
    cnt = c_ref[0]
    o_ref[0] = s / jnp.maximum(cnt, _f32(1.0))


def _finalize(sums, cnt):
    b, c, v = sums.shape
    return pl.pallas_call(
        _fin_body,
        grid=(b,),
        in_specs=[
            pl.BlockSpec((1, c, v), lambda i: (i, 0, 0)),
            pl.BlockSpec((1, 1, v), lambda i: (i, 0, 0)),
        ],
        out_specs=pl.BlockSpec((1, c, v), lambda i: (i, 0, 0)),
        out_shape=jax.ShapeDtypeStruct((b, c, v), jnp.float32),
    )(sums, cnt)


def kernel(features, coords):
    b, c, n = features.shape
    assert (b, c, n) == (_B, _C, _N), "kernel compiled for fixed shapes"
    coords = lax.stop_gradient(coords)
    norm_coords, inds3 = _prep(coords)
    sums, cnt = _get_sc_scatter()(
        features.reshape(b * c * n), inds3.reshape(b * n))
    out = _finalize(sums.reshape(b, c, NVOX), cnt.reshape(b, 1, NVOX))
    inds = lax.stop_gradient(inds3.reshape(b, n))
    return (out.reshape(b, c, RES, RES, RES), inds, norm_coords)


# speedup vs baseline: 7.7793x; 1.3158x over previous
import functools

import jax
import jax.numpy as jnp
import numpy as np
from jax import lax
from jax.experimental import pallas as pl
from jax.experimental.pallas import tpu as pltpu
from jax.experimental.pallas import tpu_sc as plsc

RES = 32
NVOX = RES ** 3

_B, _C, _N = 8, 16, 100000
_CH = 10000
_NCHUNK = _N // _CH
_GRP = _CH // 16
_CSEG = 25008
_CC0 = 12512
_CC1 = 12496
_CC1L = 12464
_IBUF = 12512


def _f32(x):
    return np.float32(x)


def _atan_poly(v):
    z = v * v
    p = _f32(8.05374449538e-2) * z - _f32(1.38776856032e-1)
    p = p * z + _f32(1.99777106478e-1)
    p = p * z - _f32(3.33329491539e-1)
    return p * z * v + v


def _atan2(y, x, signed):
    ax = jnp.abs(x)
    ay = jnp.abs(y) if signed else y
    mx = jnp.maximum(ax, ay)
    mn = jnp.minimum(ax, ay)
    big = mn > _f32(0.4142135623730951) * mx
    num = jnp.where(big, mn - mx, mn)
    den = jnp.where(big, mn + mx, mx)
    v = num / jnp.where(den > _f32(0.0), den, _f32(1.0))
    r = _atan_poly(v) + jnp.where(big, _f32(np.pi / 4), _f32(0.0))
    r = jnp.where(ay > ax, _f32(np.pi / 2) - r, r)
    r = jnp.where(x < _f32(0.0), _f32(np.pi) - r, r)
    if signed:
        r = jnp.where(y < _f32(0.0), -r, r)
    return r


def _prep_body(c_ref, nc_ref, inds_ref):
    c = c_ref[0]
    n = c.shape[1] * c.shape[2]
    mean = jnp.sum(c, axis=(1, 2), keepdims=True) * _f32(1.0 / n)
    d = c - mean
    x = d[0]
    y = d[1]
    z = d[2]
    nsq = x * x + y * y + z * z
    max_norm = jnp.sqrt(jnp.max(nsq))
    inv = _f32(1.0) / (max_norm + _f32(1e-20))
    nc = d * inv
    nc_ref[0] = nc
    xn = nc[0]
    yn = nc[1]
    zn = nc[2]
    q = xn * xn + yn * yn
    rho = jnp.sqrt(q + zn * zn)
    theta = _atan2(jnp.sqrt(q), zn, signed=False)
    phi = _atan2(yn, xn, signed=True)
    rho_bin = jnp.clip((rho * _f32(RES)).astype(jnp.int32), 0, RES - 1)
    theta_bin = jnp.clip(
        (theta / _f32(np.pi) * _f32(RES)).astype(jnp.int32), 0, RES - 1)
    phi_bin = jnp.clip(
        ((phi + _f32(np.pi)) / _f32(2.0 * np.pi) * _f32(RES)).astype(jnp.int32),
        0, RES - 1)
    inds_ref[0] = rho_bin * (RES * RES) + theta_bin * RES + phi_bin


def _prep(coords4):
    b, _, s, m = coords4.shape
    return pl.pallas_call(
        _prep_body,
        grid=(b,),
        in_specs=[pl.BlockSpec((1, 3, s, m), lambda i: (i, 0, 0, 0))],
        out_specs=[
            pl.BlockSpec((1, 3, s, m), lambda i: (i, 0, 0, 0)),
            pl.BlockSpec((1, s, m), lambda i: (i, 0, 0)),
        ],
        out_shape=[
            jax.ShapeDtypeStruct((b, 3, s, m), jnp.float32),
            jax.ShapeDtypeStruct((b, s, m), jnp.int32),
        ],
    )(coords4)


@functools.cache
def _get_sc_scatter():
    return functools.partial(
        pl.kernel,
        mesh=plsc.VectorSubcoreMesh(core_axis_name="c", subcore_axis_name="s"),
        out_type=[
            jax.ShapeDtypeStruct((_B * _C * NVOX,), jnp.float32),
            jax.ShapeDtypeStruct((32 * NVOX,), jnp.float32),
        ],
        scratch_types=[
            pltpu.VMEM((NVOX,), jnp.float32),
            pltpu.VMEM((_IBUF,), jnp.int32),
            pltpu.VMEM((_IBUF,), jnp.int32),
            pltpu.VMEM((_CH,), jnp.float32),
            pltpu.VMEM((_CH,), jnp.float32),
            pltpu.SemaphoreType.DMA,
            pltpu.SemaphoreType.DMA,
            pltpu.SemaphoreType.DMA,
        ],
        compiler_params=pltpu.CompilerParams(needs_layout_passes=False),
    )(_sc_scatter_body)


_UNROLL = 5
_ZUNROLL = 16


def _sc_scatter_body(feat_hbm, idx_hbm, sums_hbm, cnt_hbm,
                     acc, ibuf0, ibuf1, fbuf0, fbuf1, s0, s1, sw):
    wid = lax.axis_index("s") * 2 + lax.axis_index("c")
    ones = jnp.full((16,), 1.0, jnp.float32)
    zeros = jnp.zeros((16,), jnp.float32)
    ibufs, fbufs, sems = (ibuf0, ibuf1), (fbuf0, fbuf1), (s0, s1)

    def zero_acc():
        def zbody(i, carry):
            for t in range(_ZUNROLL):
                acc[pl.ds(i * (16 * _ZUNROLL) + t * 16, 16)] = zeros
            return carry
        lax.fori_loop(0, NVOX // (16 * _ZUNROLL), zbody, 0)

    def start_chunk(idx_base, feat_base, ci, p):
        st = ci * _CH
        pltpu.async_copy(
            idx_hbm.at[pl.ds(idx_base + st, _CH)],
            ibufs[p].at[pl.ds(0, _CH)], sems[p])
        pltpu.async_copy(
            feat_hbm.at[pl.ds(feat_base + st, _CH)], fbufs[p], sems[p])

    def wait_chunk(p):
        pltpu.make_async_copy(
            idx_hbm.at[pl.ds(0, _CH)], ibufs[p].at[pl.ds(0, _CH)],
            sems[p]).wait()
        pltpu.make_async_copy(
            feat_hbm.at[pl.ds(0, _CH)], fbufs[p], sems[p]).wait()

    def scatter_chunk(p):
        ib, fb = ibufs[p], fbufs[p]

        def gbody(j, carry):
            for t in range(_UNROLL):
                off = j * (16 * _UNROLL) + t * 16
                iv = ib[pl.ds(off, 16)]
                fv = fb[pl.ds(off, 16)]
                plsc.addupdate_scatter(acc, [iv], fv)
            return carry
        lax.fori_loop(0, _GRP // _UNROLL, gbody, 0)

    def wait_write():
        pltpu.make_async_copy(
            sums_hbm.at[pl.ds(0, NVOX)], acc, sw).wait()

    def run_unit(k, idx_base, feat_base, out_off):
        start_chunk(idx_base, feat_base, 0, 0)
        if k > 0:
            wait_write()
        zero_acc()

        def pair_body(i, carry):
            start_chunk(idx_base, feat_base, 2 * i + 1, 1)
            wait_chunk(0)
            scatter_chunk(0)

            @pl.when(2 * i + 2 < _NCHUNK)
            def _():
                start_chunk(idx_base, feat_base, 2 * i + 2, 0)
            wait_chunk(1)
            scatter_chunk(1)
            return carry
        lax.fori_loop(0, _NCHUNK // 2, pair_body, 0)
        pltpu.async_copy(acc, sums_hbm.at[pl.ds(out_off, NVOX)], sw)

    def cnt_start(base, off, length, p):
        pltpu.async_copy(
            idx_hbm.at[pl.ds(base + off, length)],
            ibufs[p].at[pl.ds(0, length)], sems[p])

    def cnt_wait_scatter(length, p):
        pltpu.make_async_copy(
            idx_hbm.at[pl.ds(0, length)], ibufs[p].at[pl.ds(0, length)],
            sems[p]).wait()
        ib = ibufs[p]
        npairs = length // 32

        def gbody(j, carry):
            for t in range(2):
                iv = ib[pl.ds(j * 32 + t * 16, 16)]
                plsc.addupdate_scatter(acc, [iv], ones)
            return carry
        lax.fori_loop(0, npairs, gbody, 0)
        if length % 32:
            iv = ib[pl.ds(npairs * 32, 16)]
            plsc.addupdate_scatter(acc, [iv], ones)

    def run_count():
        b = wid // 4
        p = wid % 4
        base = b * _N + p * _CSEG
        cnt_start(base, 0, _CC0, 0)
        wait_write()
        zero_acc()

        @pl.when(p < 3)
        def _():
            cnt_start(base, _CC0, _CC1, 1)

        @pl.when(p == 3)
        def _():
            cnt_start(base, _CC0, _CC1L, 1)
        cnt_wait_scatter(_CC0, 0)

        @pl.when(p < 3)
        def _():
            cnt_wait_scatter(_CC1, 1)

        @pl.when(p == 3)
        def _():
            cnt_wait_scatter(_CC1L, 1)
        pltpu.async_copy(acc, cnt_hbm.at[pl.ds(wid * NVOX, NVOX)], sw)

    for k in range(4):
        u = wid + 32 * k
        run_unit(k, (u // _C) * _N, u * _N, u * NVOX)
    run_count()
    wait_write()


def _fin_body(s_ref, c_ref, o_ref):
    s = s_ref[0]
    cnt = jnp.sum(c_ref[0], axis=0, keepdims=True)
    o_ref[0] = s / jnp.maximum(cnt, _f32(1.0))


def _finalize(sums, cnt):
    b, c, v = sums.shape
    return pl.pallas_call(
        _fin_body,
        grid=(b,),
        in_specs=[
            pl.BlockSpec((1, c, v), lambda i: (i, 0, 0)),
            pl.BlockSpec((1, 4, v), lambda i: (i, 0, 0)),
        ],
        out_specs=pl.BlockSpec((1, c, v), lambda i: (i, 0, 0)),
        out_shape=jax.ShapeDtypeStruct((b, c, v), jnp.float32),
    )(sums, cnt)


def kernel(features, coords):
    b, c, n = features.shape
    assert (b, c, n) == (_B, _C, _N), "kernel compiled for fixed shapes"
    coords = lax.stop_gradient(coords)
    nc4, inds4 = _prep(coords.reshape(b, 3, 8, n // 8))
    norm_coords = nc4.reshape(b, 3, n)
    sums, cnt = _get_sc_scatter()(
        features.reshape(b * c * n), inds4.reshape(b * n))
    out = _finalize(sums.reshape(b, c, NVOX), cnt.reshape(b, 4, NVOX))
    inds = lax.stop_gradient(inds4.reshape(b, n))
    return (out.reshape(b, c, RES, RES, RES), inds, norm_coords)
